# baseline (speedup 1.0000x reference)
import functools
import math

import jax
import jax.numpy as jnp
import numpy as np
from jax import lax
from jax.experimental import pallas as pl
from jax.experimental.pallas import tpu as pltpu
from jax.experimental.pallas import tpu_sc as plsc

_RES_MIN = 16
_RES_MAX = 256
_L = 16
_T = 65536
_B = 4
_N = _RES_MAX * _RES_MAX
_NW = 32
_PAIRS = _B * _L
_PAIRS_PER_W = _PAIRS // _NW
_C = 8192
_CHUNKS = _N // _C
_HASH_K = np.int32(-1640531535)
_IDX_MASK = np.int32(_T - 1)


_PCH = 16384
_UP = 8
_UM = 8


def _sc_body(x_hbm, cx_hbm, cy_hbm, res_hbm, feats_hbm,
             tab_v, tmp_v, cx_v, cy_v, fpk_v, res_v):
    wid = lax.axis_index("s") * 2 + lax.axis_index("c")
    pltpu.sync_copy(res_hbm, res_v)
    iota = lax.broadcasted_iota(jnp.int32, (16,), 0)

    for j in range(_PAIRS_PER_W):
        pair = wid * _PAIRS_PER_W + j
        b = pair // _L
        lvl = pair % _L
        r = res_v[pl.ds(pair * 16, 16)]

        def _pack_chunk(k, carry):
            pltpu.sync_copy(x_hbm.at[pl.ds(pair * 2 * _T + k * _PCH, _PCH)],
                            tmp_v)

            @plsc.parallel_loop(0, _PCH // 32, unroll=_UP)
            def _pk(i):
                g = i * 16
                idx = (g + iota) * 2
                ev = plsc.load_gather(tmp_v, [idx])
                od = plsc.load_gather(tmp_v, [idx + 1])
                w = plsc.bitcast(
                    plsc.pack(ev, od,
                              format=plsc.PackFormat.INTERLEAVED),
                    jnp.int32)
                tab_v[pl.ds(k * (_PCH // 2) + g, 16)] = w

            return carry

        lax.fori_loop(0, 2 * _T // _PCH, _pack_chunk, 0)

        def _main_chunk(c, carry):
            off = b * _N + c * _C
            pltpu.sync_copy(cx_hbm.at[pl.ds(off, _C)], cx_v)
            pltpu.sync_copy(cy_hbm.at[pl.ds(off, _C)], cy_v)

            @plsc.parallel_loop(0, _C // 16, unroll=_UM)
            def _body(i):
                    g = i * 16
                    sl = pl.ds(g, 16)
                    sx = cx_v[sl] * r
                    sy = cy_v[sl] * r
                    xi = sx.astype(jnp.int32)
                    yi = sy.astype(jnp.int32)
                    fx = sx - xi.astype(jnp.float32)
                    fy = sy - yi.astype(jnp.float32)
                    yk0 = yi * _HASH_K
                    yk1 = yk0 + _HASH_K
                    x1 = xi + 1
                    i00 = (xi ^ yk0) & _IDX_MASK
                    i10 = (x1 ^ yk0) & _IDX_MASK
                    i01 = (xi ^ yk1) & _IDX_MASK
                    i11 = (x1 ^ yk1) & _IDX_MASK
                    g00 = plsc.load_gather(tab_v, [i00])
                    g10 = plsc.load_gather(tab_v, [i10])
                    g01 = plsc.load_gather(tab_v, [i01])
                    g11 = plsc.load_gather(tab_v, [i11])

                    def lo(gg):
                        return lax.bitcast_convert_type(
                            jnp.left_shift(gg, 16), jnp.float32)

                    def hi(gg):
                        return lax.bitcast_convert_type(
                            gg & np.int32(-65536), jnp.float32)

                    a0 = lo(g00) + fx * (lo(g10) - lo(g00))
                    a1 = lo(g01) + fx * (lo(g11) - lo(g01))
                    b0 = hi(g00) + fx * (hi(g10) - hi(g00))
                    b1 = hi(g01) + fx * (hi(g11) - hi(g01))
                    f0 = a0 + fy * (a1 - a0)
                    f1 = b0 + fy * (b1 - b0)
                    fpk_v[sl] = plsc.bitcast(
                        plsc.pack(f0, f1,
                                  format=plsc.PackFormat.INTERLEAVED),
                        jnp.int32)

            pltpu.sync_copy(fpk_v,
                            feats_hbm.at[pl.ds(pair * _N + c * _C, _C)])
            return carry

        lax.fori_loop(0, _CHUNKS, _main_chunk, 0)


_sc_retrieve = functools.partial(
    pl.kernel,
    mesh=plsc.VectorSubcoreMesh(core_axis_name="c", subcore_axis_name="s"),
    out_type=jax.ShapeDtypeStruct((_PAIRS * _N,), jnp.int32),
    scratch_types=[
        pltpu.VMEM((_T,), jnp.int32),
        pltpu.VMEM((_PCH,), jnp.float32),
        pltpu.VMEM((_C,), jnp.float32),
        pltpu.VMEM((_C,), jnp.float32),
        pltpu.VMEM((_C,), jnp.int32),
        pltpu.VMEM((_PAIRS * 16,), jnp.float32),
    ],
    compiler_params=pltpu.CompilerParams(needs_layout_passes=False),
)(_sc_body)


_BN = 8192


def _tc_body(s_ref, awte_ref, awto_ref, abeo_ref, w8e_ref, w8o_ref,
             b8_ref, feats_ref, out_ref):
    bi = pl.program_id(0)
    srow = s_ref[pl.ds(bi, 1), :]
    se = jnp.dot(srow, awte_ref[...],
                 preferred_element_type=jnp.float32) + abeo_ref[0:1]
    so = jnp.dot(srow, awto_ref[...],
                 preferred_element_type=jnp.float32) + abeo_ref[1:2]
    we = w8e_ref[...] * se
    wo = w8o_ref[...] * so
    demod = lax.rsqrt(
        jnp.sum(we * we + wo * wo, axis=1, keepdims=True) + 1e-8)
    wed = we * demod
    wod = wo * demod

    def col(jc, carry):
        g = feats_ref[:, jc, :]
        flo = lax.bitcast_convert_type(jnp.left_shift(g, 16), jnp.float32)
        fhi = lax.bitcast_convert_type(g & np.int32(-65536), jnp.float32)
        out_ref[0, :, jc, :] = (
            jnp.dot(wed, flo, preferred_element_type=jnp.float32)
            + jnp.dot(wod, fhi, preferred_element_type=jnp.float32)
            + b8_ref[...])
        return carry

    lax.fori_loop(0, _BN // 128, col, 0)


def kernel(x, coords, s, weight, bias, affine_W, affine_b):
    b = x.shape[0]
    growth = math.exp((math.log(_RES_MAX) - math.log(_RES_MIN)) / (_L - 1))
    res = jnp.floor(_RES_MIN * growth ** jnp.arange(_L, dtype=jnp.float32))
    resx = jnp.repeat(jnp.tile(res, (_B,)), 16)

    cx = coords[..., 0].reshape(-1)
    cy = coords[..., 1].reshape(-1)
    feats = _sc_retrieve(x.reshape(-1), cx, cy,
                         resx).reshape(_PAIRS, _N // 128, 128)

    awt = affine_W.T
    abeo = jnp.stack([affine_b[0::2], affine_b[1::2]])
    w8e = jnp.zeros((8, _L), jnp.float32).at[:3].set(weight[:, 0::2])
    w8o = jnp.zeros((8, _L), jnp.float32).at[:3].set(weight[:, 1::2])
    b8 = jnp.zeros((8, 1), jnp.float32).at[:3, 0].set(bias)
    out_pad = pl.pallas_call(
        _tc_body,
        grid=(b, _N // _BN),
        in_specs=[
            pl.BlockSpec((_B, 512), lambda i, n: (0, 0)),
            pl.BlockSpec((512, _L), lambda i, n: (0, 0)),
            pl.BlockSpec((512, _L), lambda i, n: (0, 0)),
            pl.BlockSpec((2, _L), lambda i, n: (0, 0)),
            pl.BlockSpec((8, _L), lambda i, n: (0, 0)),
            pl.BlockSpec((8, _L), lambda i, n: (0, 0)),
            pl.BlockSpec((8, 1), lambda i, n: (0, 0)),
            pl.BlockSpec((_L, _BN // 128, 128), lambda i, n: (i, n, 0)),
        ],
        out_specs=pl.BlockSpec((1, 8, _BN // 128, 128),
                               lambda i, n: (i, 0, n, 0)),
        out_shape=jax.ShapeDtypeStruct((b, 8, _N // 128, 128), jnp.float32),
    )(s, awt[:, 0::2], awt[:, 1::2], abeo, w8e, w8o, b8, feats)

    return out_pad.reshape(b, 8, _N)[:, :3, :].reshape(
        b, 3, _RES_MAX, _RES_MAX)


# speedup vs baseline: 2.1223x; 2.1223x over previous
import functools
import math

import jax
import jax.numpy as jnp
import numpy as np
from jax import lax
from jax.experimental import pallas as pl
from jax.experimental.pallas import tpu as pltpu
from jax.experimental.pallas import tpu_sc as plsc

_RES_MIN = 16
_RES_MAX = 256
_L = 16
_T = 65536
_B = 4
_N = _RES_MAX * _RES_MAX
_NW = 32
_PAIRS = _B * _L
_PAIRS_PER_W = _PAIRS // _NW
_C = 8192
_CHUNKS = _N // _C
_HASH_K = np.int32(-1640531535)
_IDX_MASK = np.int32(_T - 1)


_PCH = 16384
_UP = 8
_UM = 4


def _sc_body(x_hbm, cx_hbm, cy_hbm, res_hbm, feats_hbm,
             tab_v, tmp_v, cx_v, cy_v, fpk_v, res_v):
    wid = lax.axis_index("s") * 2 + lax.axis_index("c")
    pltpu.sync_copy(res_hbm, res_v)
    iota = lax.broadcasted_iota(jnp.int32, (16,), 0)

    for j in range(_PAIRS_PER_W):
        pair = wid * _PAIRS_PER_W + j
        b = pair // _L
        lvl = pair % _L
        r = res_v[pl.ds(pair * 16, 16)]

        for k in range(2 * _T // _PCH):
            pltpu.sync_copy(x_hbm.at[pl.ds(pair * 2 * _T + k * _PCH, _PCH)],
                            tmp_v)

            @plsc.parallel_loop(0, _PCH // 32, unroll=_UP)
            def _pk(i):
                g = i * 16
                idx = (g + iota) * 2
                ev = plsc.load_gather(tmp_v, [idx])
                od = plsc.load_gather(tmp_v, [idx + 1])
                w = plsc.bitcast(
                    plsc.pack(ev, od,
                              format=plsc.PackFormat.INTERLEAVED),
                    jnp.int32)
                tab_v[pl.ds(k * (_PCH // 2) + g, 16)] = w

        for c in range(_CHUNKS):
            off = b * _N + c * _C
            pltpu.sync_copy(cx_hbm.at[pl.ds(off, _C)], cx_v)
            pltpu.sync_copy(cy_hbm.at[pl.ds(off, _C)], cy_v)

            @plsc.parallel_loop(0, _C // 16, unroll=_UM)
            def _body(i):
                    g = i * 16
                    sl = pl.ds(g, 16)
                    sx = cx_v[sl] * r
                    sy = cy_v[sl] * r
                    xi = sx.astype(jnp.int32)
                    yi = sy.astype(jnp.int32)
                    fx = sx - xi.astype(jnp.float32)
                    fy = sy - yi.astype(jnp.float32)
                    yk0 = yi * _HASH_K
                    yk1 = yk0 + _HASH_K
                    x1 = xi + 1
                    i00 = (xi ^ yk0) & _IDX_MASK
                    i10 = (x1 ^ yk0) & _IDX_MASK
                    i01 = (xi ^ yk1) & _IDX_MASK
                    i11 = (x1 ^ yk1) & _IDX_MASK
                    g00 = plsc.load_gather(tab_v, [i00])
                    g10 = plsc.load_gather(tab_v, [i10])
                    g01 = plsc.load_gather(tab_v, [i01])
                    g11 = plsc.load_gather(tab_v, [i11])

                    def lo(gg):
                        return lax.bitcast_convert_type(
                            jnp.left_shift(gg, 16), jnp.float32)

                    def hi(gg):
                        return lax.bitcast_convert_type(
                            gg & np.int32(-65536), jnp.float32)

                    a0 = lo(g00) + fx * (lo(g10) - lo(g00))
                    a1 = lo(g01) + fx * (lo(g11) - lo(g01))
                    b0 = hi(g00) + fx * (hi(g10) - hi(g00))
                    b1 = hi(g01) + fx * (hi(g11) - hi(g01))
                    f0 = a0 + fy * (a1 - a0)
                    f1 = b0 + fy * (b1 - b0)
                    fpk_v[sl] = plsc.bitcast(
                        plsc.pack(f0, f1,
                                  format=plsc.PackFormat.INTERLEAVED),
                        jnp.int32)

            pltpu.sync_copy(fpk_v,
                            feats_hbm.at[pl.ds(pair * _N + c * _C, _C)])


_sc_retrieve = functools.partial(
    pl.kernel,
    mesh=plsc.VectorSubcoreMesh(core_axis_name="c", subcore_axis_name="s"),
    out_type=jax.ShapeDtypeStruct((_PAIRS * _N,), jnp.int32),
    scratch_types=[
        pltpu.VMEM((_T,), jnp.int32),
        pltpu.VMEM((_PCH,), jnp.float32),
        pltpu.VMEM((_C,), jnp.float32),
        pltpu.VMEM((_C,), jnp.float32),
        pltpu.VMEM((_C,), jnp.int32),
        pltpu.VMEM((_PAIRS * 16,), jnp.float32),
    ],
    compiler_params=pltpu.CompilerParams(needs_layout_passes=False),
)(_sc_body)


_BN = 8192


def _tc_body(s_ref, awte_ref, awto_ref, abeo_ref, w8e_ref, w8o_ref,
             b8_ref, feats_ref, out_ref):
    bi = pl.program_id(0)
    srow = s_ref[pl.ds(bi, 1), :]
    se = jnp.dot(srow, awte_ref[...],
                 preferred_element_type=jnp.float32) + abeo_ref[0:1]
    so = jnp.dot(srow, awto_ref[...],
                 preferred_element_type=jnp.float32) + abeo_ref[1:2]
    we = w8e_ref[...] * se
    wo = w8o_ref[...] * so
    demod = lax.rsqrt(
        jnp.sum(we * we + wo * wo, axis=1, keepdims=True) + 1e-8)
    g = feats_ref[0]
    flo = lax.bitcast_convert_type(jnp.left_shift(g, 16), jnp.float32)
    fhi = lax.bitcast_convert_type(g & np.int32(-65536), jnp.float32)
    out_ref[0] = (jnp.dot(we * demod, flo,
                          preferred_element_type=jnp.float32)
                  + jnp.dot(wo * demod, fhi,
                            preferred_element_type=jnp.float32)
                  + b8_ref[...])


def kernel(x, coords, s, weight, bias, affine_W, affine_b):
    b = x.shape[0]
    growth = math.exp((math.log(_RES_MAX) - math.log(_RES_MIN)) / (_L - 1))
    res = jnp.floor(_RES_MIN * growth ** jnp.arange(_L, dtype=jnp.float32))
    resx = jnp.repeat(jnp.tile(res, (_B,)), 16)

    cx = coords[..., 0].reshape(-1)
    cy = coords[..., 1].reshape(-1)
    feats = _sc_retrieve(x.reshape(-1), cx, cy, resx).reshape(b, _L, _N)

    awt = affine_W.T
    abeo = jnp.stack([affine_b[0::2], affine_b[1::2]])
    w8e = jnp.zeros((8, _L), jnp.float32).at[:3].set(weight[:, 0::2])
    w8o = jnp.zeros((8, _L), jnp.float32).at[:3].set(weight[:, 1::2])
    b8 = jnp.zeros((8, 1), jnp.float32).at[:3, 0].set(bias)
    out_pad = pl.pallas_call(
        _tc_body,
        grid=(b, _N // _BN),
        in_specs=[
            pl.BlockSpec((_B, 512), lambda i, n: (0, 0)),
            pl.BlockSpec((512, _L), lambda i, n: (0, 0)),
            pl.BlockSpec((512, _L), lambda i, n: (0, 0)),
            pl.BlockSpec((2, _L), lambda i, n: (0, 0)),
            pl.BlockSpec((8, _L), lambda i, n: (0, 0)),
            pl.BlockSpec((8, _L), lambda i, n: (0, 0)),
            pl.BlockSpec((8, 1), lambda i, n: (0, 0)),
            pl.BlockSpec((1, _L, _BN), lambda i, n: (i, 0, n)),
        ],
        out_specs=pl.BlockSpec((1, 8, _BN), lambda i, n: (i, 0, n)),
        out_shape=jax.ShapeDtypeStruct((b, 8, _N), jnp.float32),
    )(s, awt[:, 0::2], awt[:, 1::2], abeo, w8e, w8o, b8, feats)

    return out_pad[:, :3, :].reshape(b, 3, _RES_MAX, _RES_MAX)
